# Initial kernel scaffold; baseline (speedup 1.0000x reference)
import jax
import jax.numpy as jnp
from jax.experimental import pallas as pl


def kernel(x, edge_index, W1, b1, W2, b2):
    raise NotImplementedError("write your pallas kernel here")


# speedup vs baseline: 10.6096x; 10.6096x over previous
import functools

import jax
import jax.numpy as jnp
from jax import lax
from jax.experimental import pallas as pl
from jax.experimental.pallas import tpu as pltpu
from jax.experimental.pallas import tpu_sc as plsc

N = 10000
E = 320000
IN_CH = 128
HID_CH = 256
OUT_CH = 128

NC = 2
NS = 16
NPAD = 10240
K = 80
BLK = 2048
NBLK = -(-N // BLK)

_MESH = dict(core_axis_name="c", subcore_axis_name="s")


def _make_deg_kernel():
    epw = E // (NC * NS)
    nchunks = epw // K
    rows_pt = NPAD // NS

    @functools.partial(
        pl.kernel,
        out_type=jax.ShapeDtypeStruct((NC, NPAD), jnp.float32),
        mesh=plsc.VectorSubcoreMesh(**_MESH),
        scratch_types=[
            pltpu.VMEM((1, K), jnp.int32),
            pltpu.VMEM((K,), jnp.float32),
            pltpu.VMEM_SHARED((NPAD,), jnp.float32),
        ],
    )
    def deg_kernel(dst_hbm, zeros_hbm, out_hbm, idx_v, ones_v, hist_s):
        cid = lax.axis_index("c")
        sid = lax.axis_index("s")
        for i in range(K // 16):
            ones_v[pl.ds(i * 16, 16)] = jnp.full((16,), 1.0, jnp.float32)
        r0 = pl.multiple_of(sid * rows_pt, 8)
        pltpu.sync_copy(zeros_hbm.at[pl.ds(r0, rows_pt)],
                        hist_s.at[pl.ds(r0, rows_pt)])
        plsc.subcore_barrier()
        ebase = (cid * NS + sid) * epw

        def body(k, carry):
            off = pl.multiple_of(ebase + k * K, 8)
            pltpu.sync_copy(dst_hbm.at[pl.ds(off, K)], idx_v.at[0])
            pltpu.sync_copy(ones_v, hist_s.at[idx_v.at[0]], add=True)
            return carry

        lax.fori_loop(0, nchunks, body, 0)
        plsc.subcore_barrier()
        pltpu.sync_copy(hist_s.at[pl.ds(r0, rows_pt)],
                        out_hbm.at[cid, pl.ds(r0, rows_pt)])

    return deg_kernel


RPT = 624
RTAIL = N - NS * RPT


def _make_agg_kernel(half, edge_split):
    nworkers = NC * NS if edge_split else NS
    epw = E // nworkers
    nchunks = epw // K

    @functools.partial(
        pl.kernel,
        out_type=jax.ShapeDtypeStruct((2 * N, half), jnp.float32),
        mesh=plsc.VectorSubcoreMesh(**_MESH),
        scratch_types=[
            pltpu.VMEM((1, K), jnp.int32),
            pltpu.VMEM((1, K), jnp.int32),
            pltpu.VMEM((K, half), jnp.float32),
            pltpu.VMEM_SHARED((N, half), jnp.float32),
            pltpu.SemaphoreType.DMA,
        ],
    )
    def agg_kernel(hh_hbm, src_hbm, dst_hbm, out_hbm,
                   gi_v, di_v, rows_v, acc_s, sem):
        cid = lax.axis_index("c")
        sid = lax.axis_index("s")
        row0 = pl.multiple_of(sid * RPT, 8)
        obase = cid * N
        tbase = obase if not edge_split else 0

        pltpu.sync_copy(hh_hbm.at[pl.ds(tbase + row0, RPT)],
                        acc_s.at[pl.ds(row0, RPT)])

        @pl.when(sid == NS - 1)
        def _():
            r1 = pl.multiple_of(NS * RPT, 8)
            pltpu.sync_copy(hh_hbm.at[pl.ds(tbase + r1, RTAIL)],
                            acc_s.at[pl.ds(r1, RTAIL)])

        plsc.subcore_barrier()
        ebase = (cid * NS + sid) * epw if edge_split else sid * epw

        def body(k, carry):
            off = pl.multiple_of(ebase + k * K, 8)
            pltpu.sync_copy(src_hbm.at[pl.ds(off, K)], gi_v.at[0])
            pltpu.sync_copy(dst_hbm.at[pl.ds(off, K)], di_v.at[0])
            if not edge_split:
                for i in range(K // 16):
                    s = pl.ds(i * 16, 16)
                    gi_v[0, s] = gi_v[0, s] + tbase
            pltpu.async_copy(hh_hbm.at[gi_v.at[0]], rows_v, sem).wait()
            pltpu.sync_copy(rows_v, acc_s.at[di_v.at[0]], add=True)
            return carry

        lax.fori_loop(0, nchunks, body, 0)
        plsc.subcore_barrier()
        pltpu.sync_copy(acc_s.at[pl.ds(row0, RPT)],
                        out_hbm.at[pl.ds(obase + row0, RPT)])

        @pl.when(sid == NS - 1)
        def _():
            r1 = pl.multiple_of(NS * RPT, 8)
            pltpu.sync_copy(acc_s.at[pl.ds(r1, RTAIL)],
                            out_hbm.at[pl.ds(obase + r1, RTAIL)])

    return agg_kernel


def _deg_from_hist(hist_ref):
    i = pl.program_id(0)
    hs = hist_ref[:, pl.ds(pl.multiple_of(i * BLK, 128), BLK)]
    return lax.rsqrt(hs[0, :] + hs[1, :] + 1.0)


def _mm1_body(x_ref, w_ref, hist_ref, out_ref):
    d = _deg_from_hist(hist_ref)
    h = jnp.dot(x_ref[...], w_ref[...],
                preferred_element_type=jnp.float32,
                precision=lax.Precision.HIGHEST)
    out_ref[0] = h * d[:, None]


def _mm1(x, w1, hist):
    return pl.pallas_call(
        _mm1_body,
        grid=(NBLK, 2),
        in_specs=[
            pl.BlockSpec((BLK, IN_CH), lambda i, j: (i, 0)),
            pl.BlockSpec((IN_CH, HID_CH // 2), lambda i, j: (0, j)),
            pl.BlockSpec((NC, NPAD), lambda i, j: (0, 0)),
        ],
        out_specs=pl.BlockSpec((1, BLK, HID_CH // 2), lambda i, j: (j, i, 0)),
        out_shape=jax.ShapeDtypeStruct((NC, N, HID_CH // 2), jnp.float32),
    )(x, w1, hist)


def _mm2_body(sa_ref, sb_ref, hist_ref, b1_ref, w2_ref, out_ref):
    d = _deg_from_hist(hist_ref)[:, None]
    b1 = b1_ref[...]
    za = jax.nn.relu(sa_ref[0] * d + b1[None, :HID_CH // 2])
    zb = jax.nn.relu(sb_ref[0] * d + b1[None, HID_CH // 2:])
    w2 = w2_ref[...]
    h = (jnp.dot(za, w2[:HID_CH // 2, :], preferred_element_type=jnp.float32,
                 precision=lax.Precision.HIGHEST)
         + jnp.dot(zb, w2[HID_CH // 2:, :], preferred_element_type=jnp.float32,
                   precision=lax.Precision.HIGHEST))
    out_ref[...] = h * d


def _mm2(s1, hist, b1, w2):
    return pl.pallas_call(
        _mm2_body,
        grid=(NBLK,),
        in_specs=[
            pl.BlockSpec((1, BLK, HID_CH // 2), lambda i: (0, i, 0)),
            pl.BlockSpec((1, BLK, HID_CH // 2), lambda i: (1, i, 0)),
            pl.BlockSpec((NC, NPAD), lambda i: (0, 0)),
            pl.BlockSpec((HID_CH,), lambda i: (0,)),
            pl.BlockSpec((HID_CH, OUT_CH), lambda i: (0, 0)),
        ],
        out_specs=pl.BlockSpec((BLK, OUT_CH), lambda i: (i, 0)),
        out_shape=jax.ShapeDtypeStruct((N, OUT_CH), jnp.float32),
    )(s1, s1, hist, b1, w2)


def _fin_body(p0_ref, p1_ref, h2_ref, hist_ref, b2_ref, out_ref):
    d = _deg_from_hist(hist_ref)[:, None]
    s = p0_ref[0] + p1_ref[0] - h2_ref[...]
    out_ref[...] = s * d + b2_ref[...][None, :]


def _fin(s2, h2, hist, b2):
    return pl.pallas_call(
        _fin_body,
        grid=(NBLK,),
        in_specs=[
            pl.BlockSpec((1, BLK, OUT_CH), lambda i: (0, i, 0)),
            pl.BlockSpec((1, BLK, OUT_CH), lambda i: (1, i, 0)),
            pl.BlockSpec((BLK, OUT_CH), lambda i: (i, 0)),
            pl.BlockSpec((NC, NPAD), lambda i: (0, 0)),
            pl.BlockSpec((OUT_CH,), lambda i: (0,)),
        ],
        out_specs=pl.BlockSpec((BLK, OUT_CH), lambda i: (i, 0)),
        out_shape=jax.ShapeDtypeStruct((N, OUT_CH), jnp.float32),
    )(s2, s2, h2, hist, b2)


_deg_kernel = _make_deg_kernel()
_agg_hid = _make_agg_kernel(HID_CH // 2, edge_split=False)
_agg_out = _make_agg_kernel(OUT_CH, edge_split=True)


def kernel(x, edge_index, W1, b1, W2, b2):
    src = edge_index[0]
    dst = edge_index[1]
    zeros = jnp.zeros((NPAD,), jnp.float32)
    hist = _deg_kernel(dst, zeros)
    h1 = _mm1(x, W1, hist)
    s1 = _agg_hid(h1.reshape(2 * N, HID_CH // 2), src, dst)
    h2 = _mm2(s1.reshape(NC, N, HID_CH // 2), hist, b1, W2)
    s2 = _agg_out(h2, src, dst)
    return _fin(s2.reshape(NC, N, OUT_CH), h2, hist, b2)
